# baseline (speedup 1.0000x reference)
import jax
import jax.numpy as jnp
from jax import lax
from jax.experimental import pallas as pl
from jax.experimental.pallas import tpu as pltpu
from jax.experimental.pallas import tpu_sc as plsc

N_EDGES = 320000
D = 128
N_SEG = 5
NC, NS = 2, 16
NW = NC * NS
CHUNK = 40
CPW = N_EDGES // CHUNK // NW
EPW = CHUNK * CPW
IBLK = 25
IB_EDGES = IBLK * CHUNK
NUM_RELS = 237


def _body(intra_h, node_h, src_h, dst_h, et_h, relf_h, out_h,
          src_v, dst_v, et_v, rel_v, asm0, asm1, gsem, wsem0, wsem1):
    wid = lax.axis_index("s") * NC + lax.axis_index("c")
    e0 = wid * EPW
    asms = (asm0, asm1)
    wsems = (wsem0, wsem1)
    iota = lax.broadcasted_iota(jnp.int32, (16,), 0)

    pltpu.sync_copy(relf_h, rel_v)

    def stage_rel(a, ioff):
        for g in range(3):
            nvalid = 16 if g < 2 else CHUNK - 2 * 16
            t128 = et_v[pl.ds(ioff + g * 16, 16)] * D
            row_idx = iota + (g * 16)
            mask = (iota < nvalid) if nvalid < 16 else None

            def wbody(k, carry):
                wb = jnp.full((16,), 0, jnp.int32) + k * 8
                for u in range(8):
                    idx = t128 + wb + u
                    c_vec = wb + (4 * D + u)
                    if mask is None:
                        vals = plsc.load_gather(rel_v, [idx])
                        plsc.store_scatter(a, [row_idx, c_vec], vals)
                    else:
                        vals = plsc.load_gather(rel_v, [idx], mask=mask)
                        plsc.store_scatter(a, [row_idx, c_vec], vals, mask=mask)
                return carry

            lax.fori_loop(0, D // 8, wbody, None)

    def outer(i, carry):
        for b in range(2):
            j = 2 * i + b

            @pl.when(j < CPW)
            def _():
                @pl.when(j % IBLK == 0)
                def _():
                    off = e0 + (j // IBLK) * IB_EDGES
                    pltpu.sync_copy(src_h.at[pl.ds(off, IB_EDGES)], src_v)
                    pltpu.sync_copy(dst_h.at[pl.ds(off, IB_EDGES)], dst_v)
                    pltpu.sync_copy(et_h.at[pl.ds(off, IB_EDGES)],
                                    et_v.at[pl.ds(0, IB_EDGES)])

                a = asms[b]

                @pl.when(j >= 2)
                def _():
                    pltpu.make_async_copy(
                        a, out_h.at[pl.ds(0, CHUNK)], wsems[b]).wait()

                ioff = (j % IBLK) * CHUNK
                si = src_v.at[pl.ds(ioff, CHUNK)]
                di = dst_v.at[pl.ds(ioff, CHUNK)]
                pltpu.async_copy(intra_h.at[si], a.at[:, pl.ds(0 * D, D)], gsem)
                pltpu.async_copy(intra_h.at[di], a.at[:, pl.ds(1 * D, D)], gsem)
                pltpu.async_copy(node_h.at[si], a.at[:, pl.ds(2 * D, D)], gsem)
                pltpu.async_copy(node_h.at[di], a.at[:, pl.ds(3 * D, D)], gsem)

                stage_rel(a, ioff)

                pltpu.make_async_copy(
                    out_h.at[pl.ds(0, CHUNK), pl.ds(0, 4 * D)],
                    a.at[:, pl.ds(0, 4 * D)], gsem).wait()
                pltpu.async_copy(a, out_h.at[pl.ds(e0 + j * CHUNK, CHUNK)],
                                 wsems[b])

        return carry

    lax.fori_loop(0, (CPW + 1) // 2, outer, None)
    pltpu.make_async_copy(asm0, out_h.at[pl.ds(0, CHUNK)], wsem0).wait()
    pltpu.make_async_copy(asm1, out_h.at[pl.ds(0, CHUNK)], wsem1).wait()


_gather_concat = pl.kernel(
    _body,
    out_type=jax.ShapeDtypeStruct((N_EDGES, N_SEG * D), jnp.float32),
    mesh=plsc.VectorSubcoreMesh(core_axis_name="c", subcore_axis_name="s"),
    compiler_params=pltpu.CompilerParams(needs_layout_passes=False),
    scratch_types=[
        pltpu.VMEM((IB_EDGES,), jnp.int32),
        pltpu.VMEM((IB_EDGES,), jnp.int32),
        pltpu.VMEM((IB_EDGES + 16,), jnp.int32),
        pltpu.VMEM((NUM_RELS * D,), jnp.float32),
        pltpu.VMEM((CHUNK, N_SEG * D), jnp.float32),
        pltpu.VMEM((CHUNK, N_SEG * D), jnp.float32),
        pltpu.SemaphoreType.DMA,
        pltpu.SemaphoreType.DMA,
        pltpu.SemaphoreType.DMA,
    ],
)


@jax.jit
def kernel(intra, node_repr, edge_index, edge_type, rel_emb):
    src = edge_index[0].astype(jnp.int32)
    dst = edge_index[1].astype(jnp.int32)
    et = edge_type.astype(jnp.int32)
    return _gather_concat(intra, node_repr, src, dst, et, rel_emb.reshape(-1))


# speedup vs baseline: 2.2245x; 2.2245x over previous
import jax
import jax.numpy as jnp
from jax import lax
from jax.experimental import pallas as pl
from jax.experimental.pallas import tpu as pltpu
from jax.experimental.pallas import tpu_sc as plsc

N_EDGES = 320000
D = 128
N_SEG = 5
NC, NS = 2, 16
NW = NC * NS
CHUNK = 40
CPW = N_EDGES // CHUNK // NW
EPW = CHUNK * CPW
IBLK = 25
IB_EDGES = IBLK * CHUNK
NUM_RELS = 237


def _body(intra_h, node_h, src_h, dst_h, et_h, relf_h, out_h,
          src_v, dst_v, et_v, rel_v, asm0, asm1, gsem, wsem0, wsem1):
    wid = lax.axis_index("s") * NC + lax.axis_index("c")
    e0 = wid * EPW
    asms = (asm0, asm1)
    wsems = (wsem0, wsem1)
    iota = lax.broadcasted_iota(jnp.int32, (16,), 0)

    pltpu.sync_copy(relf_h, rel_v)

    def stage_rel(a, ioff):
        for r in range(CHUNK):
            t_b = plsc.load_gather(et_v, [jnp.full((16,), ioff + r, jnp.int32)])
            base = t_b * D + iota
            for s in range(D // 16):
                vals = plsc.load_gather(rel_v, [base + s * 16])
                a[r, pl.ds(4 * D + s * 16, 16)] = vals

    def outer(i, carry):
        for b in range(2):
            j = 2 * i + b

            @pl.when(j < CPW)
            def _():
                @pl.when(j % IBLK == 0)
                def _():
                    off = e0 + (j // IBLK) * IB_EDGES
                    pltpu.sync_copy(src_h.at[pl.ds(off, IB_EDGES)], src_v)
                    pltpu.sync_copy(dst_h.at[pl.ds(off, IB_EDGES)], dst_v)
                    pltpu.sync_copy(et_h.at[pl.ds(off, IB_EDGES)], et_v)

                a = asms[b]

                @pl.when(j >= 2)
                def _():
                    pltpu.make_async_copy(
                        a, out_h.at[pl.ds(0, CHUNK)], wsems[b]).wait()

                ioff = (j % IBLK) * CHUNK
                si = src_v.at[pl.ds(ioff, CHUNK)]
                di = dst_v.at[pl.ds(ioff, CHUNK)]
                pltpu.async_copy(intra_h.at[si], a.at[:, pl.ds(0 * D, D)], gsem)
                pltpu.async_copy(intra_h.at[di], a.at[:, pl.ds(1 * D, D)], gsem)
                pltpu.async_copy(node_h.at[si], a.at[:, pl.ds(2 * D, D)], gsem)
                pltpu.async_copy(node_h.at[di], a.at[:, pl.ds(3 * D, D)], gsem)

                stage_rel(a, ioff)

                pltpu.make_async_copy(
                    out_h.at[pl.ds(0, CHUNK), pl.ds(0, 4 * D)],
                    a.at[:, pl.ds(0, 4 * D)], gsem).wait()
                pltpu.async_copy(a, out_h.at[pl.ds(e0 + j * CHUNK, CHUNK)],
                                 wsems[b])

        return carry

    lax.fori_loop(0, (CPW + 1) // 2, outer, None)
    pltpu.make_async_copy(asm0, out_h.at[pl.ds(0, CHUNK)], wsem0).wait()
    pltpu.make_async_copy(asm1, out_h.at[pl.ds(0, CHUNK)], wsem1).wait()


_gather_concat = pl.kernel(
    _body,
    out_type=jax.ShapeDtypeStruct((N_EDGES, N_SEG * D), jnp.float32),
    mesh=plsc.VectorSubcoreMesh(core_axis_name="c", subcore_axis_name="s"),
    compiler_params=pltpu.CompilerParams(needs_layout_passes=False),
    scratch_types=[
        pltpu.VMEM((IB_EDGES,), jnp.int32),
        pltpu.VMEM((IB_EDGES,), jnp.int32),
        pltpu.VMEM((IB_EDGES,), jnp.int32),
        pltpu.VMEM((NUM_RELS * D,), jnp.float32),
        pltpu.VMEM((CHUNK, N_SEG * D), jnp.float32),
        pltpu.VMEM((CHUNK, N_SEG * D), jnp.float32),
        pltpu.SemaphoreType.DMA,
        pltpu.SemaphoreType.DMA,
        pltpu.SemaphoreType.DMA,
    ],
)


@jax.jit
def kernel(intra, node_repr, edge_index, edge_type, rel_emb):
    src = edge_index[0].astype(jnp.int32)
    dst = edge_index[1].astype(jnp.int32)
    et = edge_type.astype(jnp.int32)
    return _gather_concat(intra, node_repr, src, dst, et, rel_emb.reshape(-1))


# speedup vs baseline: 2.2645x; 1.0180x over previous
import jax
import jax.numpy as jnp
from jax import lax
from jax.experimental import pallas as pl
from jax.experimental.pallas import tpu as pltpu
from jax.experimental.pallas import tpu_sc as plsc

N_EDGES = 320000
D = 128
N_SEG = 5
NC, NS = 2, 16
NW = NC * NS
CHUNK = 40
CPW = N_EDGES // CHUNK // NW
EPW = CHUNK * CPW
IBLK = 25
IB_EDGES = IBLK * CHUNK
NUM_RELS = 237


def _body(intra_h, node_h, src_h, dst_h, et_h, relf_h, out_h,
          src_v, dst_v, et_v, rel_v, asm0, asm1, gsem, wsem0, wsem1, isem):
    wid = lax.axis_index("s") * NC + lax.axis_index("c")
    e0 = wid * EPW
    asms = (asm0, asm1)
    wsems = (wsem0, wsem1)
    iota = lax.broadcasted_iota(jnp.int32, (16,), 0)

    pltpu.sync_copy(relf_h, rel_v)

    def stage_rel(a, ioff):
        for r in range(CHUNK):
            t_b = plsc.load_gather(et_v, [jnp.full((16,), ioff + r, jnp.int32)])
            base = t_b * D + iota
            for s in range(D // 16):
                vals = plsc.load_gather(rel_v, [base + s * 16])
                a[r, pl.ds(4 * D + s * 16, 16)] = vals

    def outer(i, carry):
        for b in range(2):
            j = 2 * i + b

            @pl.when(j < CPW)
            def _():
                @pl.when(j % IBLK == 0)
                def _():
                    off = e0 + (j // IBLK) * IB_EDGES
                    i0 = pltpu.async_copy(src_h.at[pl.ds(off, IB_EDGES)], src_v, isem)
                    i1 = pltpu.async_copy(dst_h.at[pl.ds(off, IB_EDGES)], dst_v, isem)
                    i2 = pltpu.async_copy(et_h.at[pl.ds(off, IB_EDGES)], et_v, isem)
                    i0.wait(); i1.wait(); i2.wait()

                a = asms[b]

                @pl.when(j >= 2)
                def _():
                    pltpu.make_async_copy(
                        a, out_h.at[pl.ds(0, CHUNK)], wsems[b]).wait()

                ioff = (j % IBLK) * CHUNK
                si = src_v.at[pl.ds(ioff, CHUNK)]
                di = dst_v.at[pl.ds(ioff, CHUNK)]
                pltpu.async_copy(intra_h.at[si], a.at[:, pl.ds(0 * D, D)], gsem)
                pltpu.async_copy(intra_h.at[di], a.at[:, pl.ds(1 * D, D)], gsem)
                pltpu.async_copy(node_h.at[si], a.at[:, pl.ds(2 * D, D)], gsem)
                pltpu.async_copy(node_h.at[di], a.at[:, pl.ds(3 * D, D)], gsem)

                stage_rel(a, ioff)

                pltpu.make_async_copy(
                    out_h.at[pl.ds(0, CHUNK), pl.ds(0, 4 * D)],
                    a.at[:, pl.ds(0, 4 * D)], gsem).wait()
                pltpu.async_copy(a, out_h.at[pl.ds(e0 + j * CHUNK, CHUNK)],
                                 wsems[b])

        return carry

    lax.fori_loop(0, (CPW + 1) // 2, outer, None)
    pltpu.make_async_copy(asm0, out_h.at[pl.ds(0, CHUNK)], wsem0).wait()
    pltpu.make_async_copy(asm1, out_h.at[pl.ds(0, CHUNK)], wsem1).wait()


_gather_concat = pl.kernel(
    _body,
    out_type=jax.ShapeDtypeStruct((N_EDGES, N_SEG * D), jnp.float32),
    mesh=plsc.VectorSubcoreMesh(core_axis_name="c", subcore_axis_name="s"),
    compiler_params=pltpu.CompilerParams(needs_layout_passes=False),
    scratch_types=[
        pltpu.VMEM((IB_EDGES,), jnp.int32),
        pltpu.VMEM((IB_EDGES,), jnp.int32),
        pltpu.VMEM((IB_EDGES,), jnp.int32),
        pltpu.VMEM((NUM_RELS * D,), jnp.float32),
        pltpu.VMEM((CHUNK, N_SEG * D), jnp.float32),
        pltpu.VMEM((CHUNK, N_SEG * D), jnp.float32),
        pltpu.SemaphoreType.DMA,
        pltpu.SemaphoreType.DMA,
        pltpu.SemaphoreType.DMA,
        pltpu.SemaphoreType.DMA,
    ],
)


@jax.jit
def kernel(intra, node_repr, edge_index, edge_type, rel_emb):
    src = edge_index[0].astype(jnp.int32)
    dst = edge_index[1].astype(jnp.int32)
    et = edge_type.astype(jnp.int32)
    return _gather_concat(intra, node_repr, src, dst, et, rel_emb.reshape(-1))


# speedup vs baseline: 2.2866x; 1.0098x over previous
import jax
import jax.numpy as jnp
from jax import lax
from jax.experimental import pallas as pl
from jax.experimental.pallas import tpu as pltpu
from jax.experimental.pallas import tpu_sc as plsc

N_EDGES = 320000
D = 128
N_SEG = 5
NC, NS = 2, 16
NW = NC * NS
CHUNK = 40
CPW = N_EDGES // CHUNK // NW
EPW = CHUNK * CPW
IBLK = 50
IB_EDGES = IBLK * CHUNK
NUM_RELS = 237


def _body(intra_h, node_h, src_h, dst_h, et_h, relf_h, out_h,
          src_v, dst_v, et_v, rel_v, asm0, asm1, gsem, wsem0, wsem1, isem):
    wid = lax.axis_index("s") * NC + lax.axis_index("c")
    e0 = wid * EPW
    asms = (asm0, asm1)
    wsems = (wsem0, wsem1)
    iota = lax.broadcasted_iota(jnp.int32, (16,), 0)

    pltpu.sync_copy(relf_h, rel_v)

    def stage_rel(a, ioff):
        for r in range(CHUNK):
            t_b = plsc.load_gather(et_v, [jnp.full((16,), ioff + r, jnp.int32)])
            base = t_b * D + iota
            for s in range(D // 16):
                vals = plsc.load_gather(rel_v, [base + s * 16])
                a[r, pl.ds(4 * D + s * 16, 16)] = vals

    def outer(i, carry):
        for b in range(2):
            j = 2 * i + b

            @pl.when(j < CPW)
            def _():
                @pl.when(j % IBLK == 0)
                def _():
                    off = e0 + (j // IBLK) * IB_EDGES
                    i0 = pltpu.async_copy(src_h.at[pl.ds(off, IB_EDGES)], src_v, isem)
                    i1 = pltpu.async_copy(dst_h.at[pl.ds(off, IB_EDGES)], dst_v, isem)
                    i2 = pltpu.async_copy(et_h.at[pl.ds(off, IB_EDGES)], et_v, isem)
                    i0.wait(); i1.wait(); i2.wait()

                a = asms[b]

                @pl.when(j >= 2)
                def _():
                    pltpu.make_async_copy(
                        a, out_h.at[pl.ds(0, CHUNK)], wsems[b]).wait()

                ioff = (j % IBLK) * CHUNK
                si = src_v.at[pl.ds(ioff, CHUNK)]
                di = dst_v.at[pl.ds(ioff, CHUNK)]
                pltpu.async_copy(intra_h.at[si], a.at[:, pl.ds(0 * D, D)], gsem)
                pltpu.async_copy(intra_h.at[di], a.at[:, pl.ds(1 * D, D)], gsem)
                pltpu.async_copy(node_h.at[si], a.at[:, pl.ds(2 * D, D)], gsem)
                pltpu.async_copy(node_h.at[di], a.at[:, pl.ds(3 * D, D)], gsem)

                stage_rel(a, ioff)

                pltpu.make_async_copy(
                    out_h.at[pl.ds(0, CHUNK), pl.ds(0, 4 * D)],
                    a.at[:, pl.ds(0, 4 * D)], gsem).wait()
                pltpu.async_copy(a, out_h.at[pl.ds(e0 + j * CHUNK, CHUNK)],
                                 wsems[b])

        return carry

    lax.fori_loop(0, (CPW + 1) // 2, outer, None)
    pltpu.make_async_copy(asm0, out_h.at[pl.ds(0, CHUNK)], wsem0).wait()
    pltpu.make_async_copy(asm1, out_h.at[pl.ds(0, CHUNK)], wsem1).wait()


_gather_concat = pl.kernel(
    _body,
    out_type=jax.ShapeDtypeStruct((N_EDGES, N_SEG * D), jnp.float32),
    mesh=plsc.VectorSubcoreMesh(core_axis_name="c", subcore_axis_name="s"),
    compiler_params=pltpu.CompilerParams(needs_layout_passes=False),
    scratch_types=[
        pltpu.VMEM((IB_EDGES,), jnp.int32),
        pltpu.VMEM((IB_EDGES,), jnp.int32),
        pltpu.VMEM((IB_EDGES,), jnp.int32),
        pltpu.VMEM((NUM_RELS * D,), jnp.float32),
        pltpu.VMEM((CHUNK, N_SEG * D), jnp.float32),
        pltpu.VMEM((CHUNK, N_SEG * D), jnp.float32),
        pltpu.SemaphoreType.DMA,
        pltpu.SemaphoreType.DMA,
        pltpu.SemaphoreType.DMA,
        pltpu.SemaphoreType.DMA,
    ],
)


@jax.jit
def kernel(intra, node_repr, edge_index, edge_type, rel_emb):
    src = edge_index[0].astype(jnp.int32)
    dst = edge_index[1].astype(jnp.int32)
    et = edge_type.astype(jnp.int32)
    return _gather_concat(intra, node_repr, src, dst, et, rel_emb.reshape(-1))


# speedup vs baseline: 2.3043x; 1.0078x over previous
import jax
import jax.numpy as jnp
from jax import lax
from jax.experimental import pallas as pl
from jax.experimental.pallas import tpu as pltpu
from jax.experimental.pallas import tpu_sc as plsc

N_EDGES = 320000
D = 128
N_SEG = 5
NC, NS = 2, 16
NW = NC * NS
CHUNK = 40
CPW = N_EDGES // CHUNK // NW
EPW = CHUNK * CPW
IBLK = 125
IB_EDGES = IBLK * CHUNK
NUM_RELS = 237


def _body(intra_h, node_h, src_h, dst_h, et_h, relf_h, out_h,
          src_v, dst_v, et_v, rel_v, asm0, asm1, gsem, wsem0, wsem1, isem):
    wid = lax.axis_index("s") * NC + lax.axis_index("c")
    e0 = wid * EPW
    asms = (asm0, asm1)
    wsems = (wsem0, wsem1)
    iota = lax.broadcasted_iota(jnp.int32, (16,), 0)

    pltpu.sync_copy(relf_h, rel_v)

    def stage_rel(a, ioff):
        for r in range(CHUNK):
            t_b = plsc.load_gather(et_v, [jnp.full((16,), ioff + r, jnp.int32)])
            base = t_b * D + iota
            for s in range(D // 16):
                vals = plsc.load_gather(rel_v, [base + s * 16])
                a[r, pl.ds(4 * D + s * 16, 16)] = vals

    def outer(i, carry):
        for b in range(2):
            j = 2 * i + b

            @pl.when(j < CPW)
            def _():
                @pl.when(j % IBLK == 0)
                def _():
                    off = e0 + (j // IBLK) * IB_EDGES
                    i0 = pltpu.async_copy(src_h.at[pl.ds(off, IB_EDGES)], src_v, isem)
                    i1 = pltpu.async_copy(dst_h.at[pl.ds(off, IB_EDGES)], dst_v, isem)
                    i2 = pltpu.async_copy(et_h.at[pl.ds(off, IB_EDGES)], et_v, isem)
                    i0.wait(); i1.wait(); i2.wait()

                a = asms[b]

                @pl.when(j >= 2)
                def _():
                    pltpu.make_async_copy(
                        a, out_h.at[pl.ds(0, CHUNK)], wsems[b]).wait()

                ioff = (j % IBLK) * CHUNK
                si = src_v.at[pl.ds(ioff, CHUNK)]
                di = dst_v.at[pl.ds(ioff, CHUNK)]
                pltpu.async_copy(intra_h.at[si], a.at[:, pl.ds(0 * D, D)], gsem)
                pltpu.async_copy(intra_h.at[di], a.at[:, pl.ds(1 * D, D)], gsem)
                pltpu.async_copy(node_h.at[si], a.at[:, pl.ds(2 * D, D)], gsem)
                pltpu.async_copy(node_h.at[di], a.at[:, pl.ds(3 * D, D)], gsem)

                stage_rel(a, ioff)

                pltpu.make_async_copy(
                    out_h.at[pl.ds(0, CHUNK), pl.ds(0, 4 * D)],
                    a.at[:, pl.ds(0, 4 * D)], gsem).wait()
                pltpu.async_copy(a, out_h.at[pl.ds(e0 + j * CHUNK, CHUNK)],
                                 wsems[b])

        return carry

    lax.fori_loop(0, (CPW + 1) // 2, outer, None)
    pltpu.make_async_copy(asm0, out_h.at[pl.ds(0, CHUNK)], wsem0).wait()
    pltpu.make_async_copy(asm1, out_h.at[pl.ds(0, CHUNK)], wsem1).wait()


_gather_concat = pl.kernel(
    _body,
    out_type=jax.ShapeDtypeStruct((N_EDGES, N_SEG * D), jnp.float32),
    mesh=plsc.VectorSubcoreMesh(core_axis_name="c", subcore_axis_name="s"),
    compiler_params=pltpu.CompilerParams(needs_layout_passes=False),
    scratch_types=[
        pltpu.VMEM((IB_EDGES,), jnp.int32),
        pltpu.VMEM((IB_EDGES,), jnp.int32),
        pltpu.VMEM((IB_EDGES,), jnp.int32),
        pltpu.VMEM((NUM_RELS * D,), jnp.float32),
        pltpu.VMEM((CHUNK, N_SEG * D), jnp.float32),
        pltpu.VMEM((CHUNK, N_SEG * D), jnp.float32),
        pltpu.SemaphoreType.DMA,
        pltpu.SemaphoreType.DMA,
        pltpu.SemaphoreType.DMA,
        pltpu.SemaphoreType.DMA,
    ],
)


@jax.jit
def kernel(intra, node_repr, edge_index, edge_type, rel_emb):
    src = edge_index[0].astype(jnp.int32)
    dst = edge_index[1].astype(jnp.int32)
    et = edge_type.astype(jnp.int32)
    return _gather_concat(intra, node_repr, src, dst, et, rel_emb.reshape(-1))
